# Initial kernel scaffold; baseline (speedup 1.0000x reference)
import jax
import jax.numpy as jnp
from jax.experimental import pallas as pl


def kernel(X, family_table, store_table, W1, b1, g1, be1, W2, b2, g2, be2, W3, b3):
    raise NotImplementedError("write your pallas kernel here")


# speedup vs baseline: 2.8252x; 2.8252x over previous
import jax
import jax.numpy as jnp
from jax.experimental import pallas as pl
from jax.experimental.pallas import tpu as pltpu

B = 16384
EPS = 1e-5


def _mlp_body(X_ref, ft_ref, st_ref, W1a_ref, W1b_ref, W1c_ref, b1_ref,
              g1_ref, be1_ref, W2_ref, b2_ref, g2_ref, be2_ref,
              W3_ref, b3_ref, out_ref):
    X = X_ref[...]
    sidx = X[:, 8:9].astype(jnp.int32)
    fidx = X[:, 9:10].astype(jnp.int32)

    iota_s = jax.lax.broadcasted_iota(jnp.int32, (B, 54), 1)
    iota_f = jax.lax.broadcasted_iota(jnp.int32, (B, 33), 1)
    oh_s = (iota_s == sidx).astype(jnp.float32)
    oh_f = (iota_f == fidx).astype(jnp.float32)

    ftW = jnp.dot(ft_ref[...], W1b_ref[...], preferred_element_type=jnp.float32)
    stW = jnp.dot(st_ref[...], W1c_ref[...], preferred_element_type=jnp.float32)
    h = (jnp.dot(X[:, 0:8], W1a_ref[...], preferred_element_type=jnp.float32)
         + jnp.dot(oh_f, ftW, preferred_element_type=jnp.float32)
         + jnp.dot(oh_s, stW, preferred_element_type=jnp.float32)
         + b1_ref[...])
    h = jnp.maximum(h, 0.0)
    mu = jnp.mean(h, axis=0, keepdims=True)
    var = jnp.mean((h - mu) * (h - mu), axis=0, keepdims=True)
    h = g1_ref[...] * (h - mu) * jax.lax.rsqrt(var + EPS) + be1_ref[...]

    h = jnp.dot(h, W2_ref[...], preferred_element_type=jnp.float32) + b2_ref[...]
    h = jnp.maximum(h, 0.0)
    mu2 = jnp.mean(h, axis=0, keepdims=True)
    var2 = jnp.mean((h - mu2) * (h - mu2), axis=0, keepdims=True)
    h = g2_ref[...] * (h - mu2) * jax.lax.rsqrt(var2 + EPS) + be2_ref[...]

    out_ref[...] = (jnp.dot(h, W3_ref[...], preferred_element_type=jnp.float32)
                    + b3_ref[...])


def kernel(X, family_table, store_table, W1, b1, g1, be1, W2, b2, g2, be2, W3, b3):
    W1a = W1[0:8]
    W1b = W1[8:23]
    W1c = W1[23:38]
    args = (X, family_table, store_table, W1a, W1b, W1c,
            b1.reshape(1, -1), g1.reshape(1, -1), be1.reshape(1, -1),
            W2, b2.reshape(1, -1), g2.reshape(1, -1), be2.reshape(1, -1),
            W3, b3.reshape(1, -1))
    return pl.pallas_call(
        _mlp_body,
        out_shape=jax.ShapeDtypeStruct((B, 1), jnp.float32),
    )(*args)
